# Initial kernel scaffold; baseline (speedup 1.0000x reference)
import jax
import jax.numpy as jnp
from jax.experimental import pallas as pl


def kernel(x, edge_index, W0, b0, W1, b1):
    raise NotImplementedError("write your pallas kernel here")


# speedup vs baseline: 35.7524x; 35.7524x over previous
import functools

import jax
import jax.numpy as jnp
from jax import lax
from jax.experimental import pallas as pl
from jax.experimental.pallas import tpu as pltpu
from jax.experimental.pallas import tpu_sc as plsc

N = 10000
D = 128
DG = 64
E = 320000

NC = 2
NS = 16
NT = NC * NS
NP = 10240
RT = NP // NS
EP = 327680
ET = EP // NT
K = 128
C = ET // K
PAD_ROWS = NP - N

f32 = jnp.float32

_mesh = plsc.VectorSubcoreMesh(
    core_axis_name="c", subcore_axis_name="s", num_cores=NC, num_subcores=NS)


@functools.partial(
    pl.kernel,
    out_type=jax.ShapeDtypeStruct((NC, NP), f32),
    mesh=_mesh,
    scratch_types=[
        pltpu.VMEM((C, K), jnp.int32),
        pltpu.VMEM((K,), f32),
        pltpu.VMEM((RT,), f32),
        pltpu.VMEM_SHARED((NP,), f32),
    ],
    compiler_params=pltpu.CompilerParams(use_tc_tiling_on_sc=False),
)
def _deg_call(dst_hbm, ones_hbm, zrow_hbm, out_hbm, dst_v, ones_v, z_v, deg_sh):
    c = lax.axis_index("c")
    s = lax.axis_index("s")
    wid = s * NC + c
    rs = s * RT
    pltpu.sync_copy(dst_hbm.at[wid], dst_v)
    pltpu.sync_copy(ones_hbm, ones_v)
    pltpu.sync_copy(zrow_hbm, z_v)
    pltpu.sync_copy(z_v, deg_sh.at[pl.ds(rs, RT)])
    plsc.subcore_barrier()

    def body(i, carry):
        pltpu.sync_copy(ones_v, deg_sh.at[dst_v.at[i]], add=True)
        return carry

    lax.fori_loop(0, C, body, 0)
    plsc.subcore_barrier()
    pltpu.sync_copy(deg_sh.at[pl.ds(rs, RT)], out_hbm.at[c, pl.ds(rs, RT)])


@functools.partial(
    pl.kernel,
    out_type=jax.ShapeDtypeStruct((NC, NP, DG), f32),
    mesh=_mesh,
    scratch_types=[
        pltpu.VMEM((C, K), jnp.int32),
        pltpu.VMEM((C, K), jnp.int32),
        pltpu.VMEM((K, DG), f32),
        pltpu.VMEM((K, DG), f32),
        pltpu.VMEM_SHARED((NP, DG), f32),
        pltpu.SemaphoreType.DMA,
        pltpu.SemaphoreType.DMA,
    ],
    compiler_params=pltpu.CompilerParams(use_tc_tiling_on_sc=False),
)
def _scat_call(g_hbm, src_hbm, dst_hbm, zrows_hbm, out_hbm,
               src_v, dst_v, rows0, rows1, acc_sh, sem0, sem1):
    c = lax.axis_index("c")
    s = lax.axis_index("s")
    wid = s * NC + c
    rs = s * RT
    pltpu.sync_copy(src_hbm.at[wid], src_v)
    pltpu.sync_copy(dst_hbm.at[wid], dst_v)

    @pl.when(c == 0)
    def _():
        pltpu.sync_copy(g_hbm.at[pl.ds(rs, RT)], acc_sh.at[pl.ds(rs, RT)])

    @pl.when(c != 0)
    def _():
        pltpu.sync_copy(zrows_hbm, acc_sh.at[pl.ds(rs, RT)])

    plsc.subcore_barrier()

    pltpu.async_copy(g_hbm.at[src_v.at[0]], rows0, sem0)
    pltpu.async_copy(g_hbm.at[src_v.at[1]], rows1, sem1)

    def body(i, carry):
        for b, (rows_b, sem_b) in enumerate(((rows0, sem0), (rows1, sem1))):
            ch = 2 * i + b
            pltpu.make_async_copy(g_hbm.at[src_v.at[ch]], rows_b, sem_b).wait()
            pltpu.sync_copy(rows_b, acc_sh.at[dst_v.at[ch]], add=True)

            @pl.when(ch + 2 < C)
            def _():
                pltpu.async_copy(g_hbm.at[src_v.at[ch + 2]], rows_b, sem_b)

        return carry

    lax.fori_loop(0, C // 2, body, 0)
    plsc.subcore_barrier()
    pltpu.sync_copy(acc_sh.at[pl.ds(rs, RT)], out_hbm.at[c, pl.ds(rs, RT)])


def _tc1_body(degT_ref, x_ref, W0_ref, b0_ref, g0_ref, dinv_ref):
    deg = degT_ref[:, 0:1] + degT_ref[:, 1:2] + 1.0
    dinv = lax.rsqrt(deg)
    h = jnp.dot(x_ref[:, DG:], W0_ref[...],
                preferred_element_type=f32) + b0_ref[...]
    g0_ref[...] = h * dinv
    dinv_ref[...] = dinv


def _tc2_body(x_ref, acc_ref, dinv_ref, W1_ref, b1_ref, y0_ref, g1_ref):
    accs = acc_ref[0] + acc_ref[1]
    fm = jnp.maximum(accs * dinv_ref[...], 0.0)
    y0 = x_ref[:, :DG] + fm
    y0_ref[...] = y0
    h1 = jnp.dot(y0, W1_ref[...], preferred_element_type=f32) + b1_ref[...]
    g1_ref[...] = h1 * dinv_ref[...]


def _tc3_body(x_ref, acc_ref, dinv_ref, y0_ref, out_ref):
    accs = acc_ref[0] + acc_ref[1]
    fm = jnp.maximum(accs * dinv_ref[...], 0.0)
    out_ref[:, :DG] = y0_ref[...]
    out_ref[:, DG:] = x_ref[:, DG:] + fm


_tc1 = pl.pallas_call(
    _tc1_body,
    out_shape=[jax.ShapeDtypeStruct((NP, DG), f32),
               jax.ShapeDtypeStruct((NP, 1), f32)],
)

_tc2 = pl.pallas_call(
    _tc2_body,
    out_shape=[jax.ShapeDtypeStruct((NP, DG), f32),
               jax.ShapeDtypeStruct((NP, DG), f32)],
)

_tc3 = pl.pallas_call(
    _tc3_body,
    out_shape=jax.ShapeDtypeStruct((NP, D), f32),
)


def kernel(x, edge_index, W0, b0, W1, b1):
    xp = jnp.pad(x.astype(f32), ((0, NP - N), (0, 0)))
    src = edge_index[0].astype(jnp.int32)
    dst = edge_index[1].astype(jnp.int32)
    npad = EP - E
    pad_idx = N + (jnp.arange(npad, dtype=jnp.int32) % PAD_ROWS)
    srcp = jnp.concatenate([src, pad_idx]).reshape(NT, C, K)
    dstp = jnp.concatenate([dst, pad_idx]).reshape(NT, C, K)

    ones_k = jnp.ones((K,), f32)
    zrow = jnp.zeros((RT,), f32)
    zrows = jnp.zeros((RT, DG), f32)

    degp = _deg_call(dstp, ones_k, zrow)
    degT = degp.T

    g0, dinv = _tc1(degT, xp, W0, b0.reshape(1, DG))
    acc0 = _scat_call(g0, srcp, dstp, zrows)
    y0, g1 = _tc2(xp, acc0, dinv, W1, b1.reshape(1, DG))
    acc1 = _scat_call(g1, srcp, dstp, zrows)
    out = _tc3(xp, acc1, dinv, y0)
    return out[:N]


# speedup vs baseline: 39.5936x; 1.1074x over previous
import functools

import jax
import jax.numpy as jnp
from jax import lax
from jax.experimental import pallas as pl
from jax.experimental.pallas import tpu as pltpu
from jax.experimental.pallas import tpu_sc as plsc

N = 10000
D = 128
DG = 64
E = 320000

NC = 2
NS = 16
NT = NC * NS
NP = 10240
RT = NP // NS
EP = 327680
ET = EP // NT
K = 128
C = ET // K
PAD_ROWS = NP - N

f32 = jnp.float32

_mesh = plsc.VectorSubcoreMesh(
    core_axis_name="c", subcore_axis_name="s", num_cores=NC, num_subcores=NS)


@functools.partial(
    pl.kernel,
    out_type=jax.ShapeDtypeStruct((NC, NP), f32),
    mesh=_mesh,
    scratch_types=[
        pltpu.VMEM((C, K), jnp.int32),
        pltpu.VMEM((K,), f32),
        pltpu.VMEM((RT,), f32),
        pltpu.VMEM_SHARED((NP,), f32),
    ],
    compiler_params=pltpu.CompilerParams(use_tc_tiling_on_sc=False),
)
def _deg_call(dst_hbm, ones_hbm, zrow_hbm, out_hbm, dst_v, ones_v, z_v, deg_sh):
    c = lax.axis_index("c")
    s = lax.axis_index("s")
    wid = s * NC + c
    rs = s * RT
    pltpu.sync_copy(dst_hbm.at[wid], dst_v)
    pltpu.sync_copy(ones_hbm, ones_v)
    pltpu.sync_copy(zrow_hbm, z_v)
    pltpu.sync_copy(z_v, deg_sh.at[pl.ds(rs, RT)])
    plsc.subcore_barrier()

    def body(i, carry):
        pltpu.sync_copy(ones_v, deg_sh.at[dst_v.at[i]], add=True)
        return carry

    lax.fori_loop(0, C, body, 0)
    plsc.subcore_barrier()
    pltpu.sync_copy(deg_sh.at[pl.ds(rs, RT)], out_hbm.at[c, pl.ds(rs, RT)])


@functools.partial(
    pl.kernel,
    out_type=jax.ShapeDtypeStruct((NC, NP, DG), f32),
    mesh=_mesh,
    scratch_types=[
        pltpu.VMEM((C, K), jnp.int32),
        pltpu.VMEM((C, K), jnp.int32),
        pltpu.VMEM((K, DG), f32),
        pltpu.VMEM((K, DG), f32),
        pltpu.VMEM((K, DG), f32),
        pltpu.VMEM((K, DG), f32),
        pltpu.VMEM_SHARED((NP, DG), f32),
        pltpu.SemaphoreType.DMA,
        pltpu.SemaphoreType.DMA,
        pltpu.SemaphoreType.DMA,
        pltpu.SemaphoreType.DMA,
        pltpu.SemaphoreType.DMA,
        pltpu.SemaphoreType.DMA,
        pltpu.SemaphoreType.DMA,
        pltpu.SemaphoreType.DMA,
    ],
    compiler_params=pltpu.CompilerParams(use_tc_tiling_on_sc=False),
)
def _scat_call(g_hbm, src_hbm, dst_hbm, zrows_hbm, out_hbm,
               src_v, dst_v, rows0, rows1, rows2, rows3, acc_sh,
               gs0, gs1, gs2, gs3, ss0, ss1, ss2, ss3):
    c = lax.axis_index("c")
    s = lax.axis_index("s")
    wid = s * NC + c
    rs = s * RT
    rows = (rows0, rows1, rows2, rows3)
    gsem = (gs0, gs1, gs2, gs3)
    ssem = (ss0, ss1, ss2, ss3)
    pltpu.sync_copy(src_hbm.at[wid], src_v)
    pltpu.sync_copy(dst_hbm.at[wid], dst_v)

    @pl.when(c == 0)
    def _():
        pltpu.sync_copy(g_hbm.at[pl.ds(rs, RT)], acc_sh.at[pl.ds(rs, RT)])

    @pl.when(c != 0)
    def _():
        pltpu.sync_copy(zrows_hbm, acc_sh.at[pl.ds(rs, RT)])

    plsc.subcore_barrier()

    pltpu.async_copy(g_hbm.at[src_v.at[0]], rows0, gs0)
    pltpu.async_copy(g_hbm.at[src_v.at[1]], rows1, gs1)

    def body(i, carry):
        for b in range(4):
            ch = 4 * i + b
            bb = (b + 2) % 4

            @pl.when(ch + 2 < C)
            def _():
                @pl.when(ch >= 2)
                def _():
                    pltpu.make_async_copy(
                        rows[bb], acc_sh.at[dst_v.at[ch - 2]], ssem[bb]).wait()

                pltpu.async_copy(g_hbm.at[src_v.at[ch + 2]], rows[bb], gsem[bb])

            pltpu.make_async_copy(g_hbm.at[src_v.at[ch]], rows[b], gsem[b]).wait()
            pltpu.async_copy(rows[b], acc_sh.at[dst_v.at[ch]], ssem[b], add=True)

        return carry

    lax.fori_loop(0, C // 4, body, 0)
    for ch in range(C - 4, C):
        b = ch % 4
        pltpu.make_async_copy(rows[b], acc_sh.at[dst_v.at[ch]], ssem[b]).wait()
    plsc.subcore_barrier()
    pltpu.sync_copy(acc_sh.at[pl.ds(rs, RT)], out_hbm.at[c, pl.ds(rs, RT)])


def _tc1a_body(x_ref, W0_ref, b0_ref, h0_ref):
    h0_ref[...] = jnp.dot(x_ref[:, DG:], W0_ref[...],
                          preferred_element_type=f32) + b0_ref[...]


def _tc1b_body(degT_ref, h0_ref, g0_ref, dinv_ref):
    deg = degT_ref[:, 0:1] + degT_ref[:, 1:2] + 1.0
    dinv = lax.rsqrt(deg)
    g0_ref[...] = h0_ref[...] * dinv
    dinv_ref[...] = dinv


def _tc2_body(x_ref, acc_ref, dinv_ref, W1_ref, b1_ref, y0_ref, g1_ref):
    accs = acc_ref[0] + acc_ref[1]
    fm = jnp.maximum(accs * dinv_ref[...], 0.0)
    y0 = x_ref[:, :DG] + fm
    y0_ref[...] = y0
    h1 = jnp.dot(y0, W1_ref[...], preferred_element_type=f32) + b1_ref[...]
    g1_ref[...] = h1 * dinv_ref[...]


def _tc3_body(x_ref, acc_ref, dinv_ref, y0_ref, out_ref):
    accs = acc_ref[0, :N] + acc_ref[1, :N]
    fm = jnp.maximum(accs * dinv_ref[:N], 0.0)
    out_ref[:, :DG] = y0_ref[:N]
    out_ref[:, DG:] = x_ref[:N, DG:] + fm


_tc1a = pl.pallas_call(
    _tc1a_body,
    out_shape=jax.ShapeDtypeStruct((NP, DG), f32),
)

_tc1b = pl.pallas_call(
    _tc1b_body,
    out_shape=[jax.ShapeDtypeStruct((NP, DG), f32),
               jax.ShapeDtypeStruct((NP, 1), f32)],
)

_tc2 = pl.pallas_call(
    _tc2_body,
    out_shape=[jax.ShapeDtypeStruct((NP, DG), f32),
               jax.ShapeDtypeStruct((NP, DG), f32)],
)

_tc3 = pl.pallas_call(
    _tc3_body,
    out_shape=jax.ShapeDtypeStruct((N, D), f32),
)


def kernel(x, edge_index, W0, b0, W1, b1):
    xp = jnp.pad(x.astype(f32), ((0, NP - N), (0, 0)))
    src = edge_index[0].astype(jnp.int32)
    dst = edge_index[1].astype(jnp.int32)
    npad = EP - E
    pad_idx = N + (jnp.arange(npad, dtype=jnp.int32) % PAD_ROWS)
    srcp = jnp.concatenate([src, pad_idx]).reshape(NT, C, K)
    dstp = jnp.concatenate([dst, pad_idx]).reshape(NT, C, K)

    ones_k = jnp.ones((K,), f32)
    zrow = jnp.zeros((RT,), f32)
    zrows = jnp.zeros((RT, DG), f32)

    degp = _deg_call(dstp, ones_k, zrow)
    degT = degp.T

    h0 = _tc1a(xp, W0, b0.reshape(1, DG))
    g0, dinv = _tc1b(degT, h0)
    acc0 = _scat_call(g0, srcp, dstp, zrows)
    y0, g1 = _tc2(xp, acc0, dinv, W1, b1.reshape(1, DG))
    acc1 = _scat_call(g1, srcp, dstp, zrows)
    return _tc3(xp, acc1, dinv, y0)


# speedup vs baseline: 48.0824x; 1.2144x over previous
import functools

import jax
import jax.numpy as jnp
from jax import lax
from jax.experimental import pallas as pl
from jax.experimental.pallas import tpu as pltpu
from jax.experimental.pallas import tpu_sc as plsc

N = 10000
D = 128
DG = 64
E = 320000

NC = 2
NS = 16
NT = NC * NS
NP = 10240
RT = NP // NS
EP = 327680
ET = EP // NT
K = 128
C = ET // K
PAD_ROWS = NP - N

f32 = jnp.float32
bf16 = jnp.bfloat16

_mesh = plsc.VectorSubcoreMesh(
    core_axis_name="c", subcore_axis_name="s", num_cores=NC, num_subcores=NS)


@functools.partial(
    pl.kernel,
    out_type=jax.ShapeDtypeStruct((NC, NP), f32),
    mesh=_mesh,
    scratch_types=[
        pltpu.VMEM((C, K), jnp.int32),
        pltpu.VMEM((K,), f32),
        pltpu.VMEM((RT,), f32),
        pltpu.VMEM_SHARED((NP,), f32),
    ],
    compiler_params=pltpu.CompilerParams(use_tc_tiling_on_sc=False),
)
def _deg_call(dst_hbm, ones_hbm, zrow_hbm, out_hbm, dst_v, ones_v, z_v, deg_sh):
    c = lax.axis_index("c")
    s = lax.axis_index("s")
    wid = s * NC + c
    rs = s * RT
    pltpu.sync_copy(dst_hbm.at[wid], dst_v)
    pltpu.sync_copy(ones_hbm, ones_v)
    pltpu.sync_copy(zrow_hbm, z_v)
    pltpu.sync_copy(z_v, deg_sh.at[pl.ds(rs, RT)])
    plsc.subcore_barrier()

    def body(i, carry):
        pltpu.sync_copy(ones_v, deg_sh.at[dst_v.at[i]], add=True)
        return carry

    lax.fori_loop(0, C, body, 0)
    plsc.subcore_barrier()
    pltpu.sync_copy(deg_sh.at[pl.ds(rs, RT)], out_hbm.at[c, pl.ds(rs, RT)])


@functools.partial(
    pl.kernel,
    out_type=jax.ShapeDtypeStruct((NC, NP, DG), bf16),
    mesh=_mesh,
    scratch_types=[
        pltpu.VMEM((C, K), jnp.int32),
        pltpu.VMEM((C, K), jnp.int32),
        pltpu.VMEM((K, DG), bf16),
        pltpu.VMEM((K, DG), bf16),
        pltpu.VMEM((K, DG), bf16),
        pltpu.VMEM((K, DG), bf16),
        pltpu.VMEM_SHARED((NP, DG), bf16),
        pltpu.SemaphoreType.DMA,
        pltpu.SemaphoreType.DMA,
        pltpu.SemaphoreType.DMA,
        pltpu.SemaphoreType.DMA,
        pltpu.SemaphoreType.DMA,
        pltpu.SemaphoreType.DMA,
        pltpu.SemaphoreType.DMA,
        pltpu.SemaphoreType.DMA,
    ],
    compiler_params=pltpu.CompilerParams(use_tc_tiling_on_sc=False),
)
def _scat_call(g_hbm, src_hbm, dst_hbm, zrows_hbm, out_hbm,
               src_v, dst_v, rows0, rows1, rows2, rows3, acc_sh,
               gs0, gs1, gs2, gs3, ss0, ss1, ss2, ss3):
    c = lax.axis_index("c")
    s = lax.axis_index("s")
    wid = s * NC + c
    rs = s * RT
    rows = (rows0, rows1, rows2, rows3)
    gsem = (gs0, gs1, gs2, gs3)
    ssem = (ss0, ss1, ss2, ss3)
    pltpu.sync_copy(src_hbm.at[wid], src_v)
    pltpu.sync_copy(dst_hbm.at[wid], dst_v)

    @pl.when(c == 0)
    def _():
        pltpu.sync_copy(g_hbm.at[pl.ds(rs, RT)], acc_sh.at[pl.ds(rs, RT)])

    @pl.when(c != 0)
    def _():
        pltpu.sync_copy(zrows_hbm, acc_sh.at[pl.ds(rs, RT)])

    plsc.subcore_barrier()

    pltpu.async_copy(g_hbm.at[src_v.at[0]], rows0, gs0)
    pltpu.async_copy(g_hbm.at[src_v.at[1]], rows1, gs1)

    def body(i, carry):
        for b in range(4):
            ch = 4 * i + b
            bb = (b + 2) % 4

            @pl.when(ch + 2 < C)
            def _():
                @pl.when(ch >= 2)
                def _():
                    pltpu.make_async_copy(
                        rows[bb], acc_sh.at[dst_v.at[ch - 2]], ssem[bb]).wait()

                pltpu.async_copy(g_hbm.at[src_v.at[ch + 2]], rows[bb], gsem[bb])

            pltpu.make_async_copy(g_hbm.at[src_v.at[ch]], rows[b], gsem[b]).wait()
            pltpu.async_copy(rows[b], acc_sh.at[dst_v.at[ch]], ssem[b], add=True)

        return carry

    lax.fori_loop(0, C // 4, body, 0)
    for ch in range(C - 4, C):
        b = ch % 4
        pltpu.make_async_copy(rows[b], acc_sh.at[dst_v.at[ch]], ssem[b]).wait()
    plsc.subcore_barrier()
    pltpu.sync_copy(acc_sh.at[pl.ds(rs, RT)], out_hbm.at[c, pl.ds(rs, RT)])


def _tc1a_body(x_ref, W0_ref, b0_ref, h0_ref):
    h0_ref[...] = jnp.dot(x_ref[:, DG:], W0_ref[...],
                          preferred_element_type=f32) + b0_ref[...]


def _tc1b_body(degT_ref, h0_ref, g0_ref, dinv_ref):
    deg = degT_ref[:, 0:1] + degT_ref[:, 1:2] + 1.0
    dinv = lax.rsqrt(deg)
    g0_ref[...] = (h0_ref[...] * dinv).astype(bf16)
    dinv_ref[...] = dinv


def _tc2_body(x_ref, acc_ref, dinv_ref, W1_ref, b1_ref, y0_ref, g1_ref):
    accs = acc_ref[0].astype(f32) + acc_ref[1].astype(f32)
    fm = jnp.maximum(accs * dinv_ref[...], 0.0)
    y0 = x_ref[:, :DG] + fm
    y0_ref[...] = y0
    h1 = jnp.dot(y0, W1_ref[...], preferred_element_type=f32) + b1_ref[...]
    g1_ref[...] = (h1 * dinv_ref[...]).astype(bf16)


def _tc3_body(x_ref, acc_ref, dinv_ref, y0_ref, out_ref):
    accs = acc_ref[0, :N].astype(f32) + acc_ref[1, :N].astype(f32)
    fm = jnp.maximum(accs * dinv_ref[:N], 0.0)
    out_ref[:, :DG] = y0_ref[:N]
    out_ref[:, DG:] = x_ref[:N, DG:] + fm


_tc1a = pl.pallas_call(
    _tc1a_body,
    out_shape=jax.ShapeDtypeStruct((NP, DG), f32),
)

_tc1b = pl.pallas_call(
    _tc1b_body,
    out_shape=[jax.ShapeDtypeStruct((NP, DG), bf16),
               jax.ShapeDtypeStruct((NP, 1), f32)],
)

_tc2 = pl.pallas_call(
    _tc2_body,
    out_shape=[jax.ShapeDtypeStruct((NP, DG), f32),
               jax.ShapeDtypeStruct((NP, DG), bf16)],
)

_tc3 = pl.pallas_call(
    _tc3_body,
    out_shape=jax.ShapeDtypeStruct((N, D), f32),
)


def kernel(x, edge_index, W0, b0, W1, b1):
    xp = jnp.pad(x.astype(f32), ((0, NP - N), (0, 0)))
    src = edge_index[0].astype(jnp.int32)
    dst = edge_index[1].astype(jnp.int32)
    npad = EP - E
    pad_idx = N + (jnp.arange(npad, dtype=jnp.int32) % PAD_ROWS)
    srcp = jnp.concatenate([src, pad_idx]).reshape(NT, C, K)
    dstp = jnp.concatenate([dst, pad_idx]).reshape(NT, C, K)

    ones_k = jnp.ones((K,), f32)
    zrow = jnp.zeros((RT,), f32)
    zrows = jnp.zeros((RT, DG), bf16)

    degp = _deg_call(dstp, ones_k, zrow)
    degT = degp.T

    h0 = _tc1a(xp, W0, b0.reshape(1, DG))
    g0, dinv = _tc1b(degT, h0)
    acc0 = _scat_call(g0, srcp, dstp, zrows)
    y0, g1 = _tc2(xp, acc0, dinv, W1, b1.reshape(1, DG))
    acc1 = _scat_call(g1, srcp, dstp, zrows)
    return _tc3(xp, acc1, dinv, y0)


# speedup vs baseline: 53.4351x; 1.1113x over previous
import functools

import jax
import jax.numpy as jnp
from jax import lax
from jax.experimental import pallas as pl
from jax.experimental.pallas import tpu as pltpu
from jax.experimental.pallas import tpu_sc as plsc

N = 10000
D = 128
DG = 64
E = 320000

NC = 2
NS = 16
NT = NC * NS
NP = 10240
RT = NP // NS
K = 128
TCH = E // K
CB = TCH // NT
REM = TCH - CB * NT
C = CB + 2
PAD_ROWS = NP - N

f32 = jnp.float32
bf16 = jnp.bfloat16

_mesh = plsc.VectorSubcoreMesh(
    core_axis_name="c", subcore_axis_name="s", num_cores=NC, num_subcores=NS)


def _stage_slab(ei_hbm, which, slab_v, wid):
    base = wid * CB + jnp.minimum(wid, REM)
    pltpu.sync_copy(ei_hbm.at[which, pl.ds(base, CB)], slab_v.at[pl.ds(0, CB)])

    @pl.when(wid < REM)
    def _():
        pltpu.sync_copy(ei_hbm.at[which, pl.ds(base + CB, 1)],
                        slab_v.at[pl.ds(CB, 1)])

    def fill_row(r):
        for j in range(K // 16):
            vec = N + ((wid * K + r * 64 + j * 16
                        + lax.iota(jnp.int32, 16)) % PAD_ROWS)
            slab_v[r, pl.ds(j * 16, 16)] = vec

    @pl.when(wid >= REM)
    def _():
        fill_row(CB)

    fill_row(CB + 1)


@functools.partial(
    pl.kernel,
    out_type=jax.ShapeDtypeStruct((NC, NP), f32),
    mesh=_mesh,
    scratch_types=[
        pltpu.VMEM((C, K), jnp.int32),
        pltpu.VMEM((K,), f32),
        pltpu.VMEM((RT,), f32),
        pltpu.VMEM_SHARED((NP,), f32),
        pltpu.SemaphoreType.DMA,
    ],
    compiler_params=pltpu.CompilerParams(use_tc_tiling_on_sc=False),
)
def _deg_call(ei_hbm, ones_hbm, zrow_hbm, out_hbm, dst_v, ones_v, z_v,
              deg_sh, ssem):
    c = lax.axis_index("c")
    s = lax.axis_index("s")
    wid = s * NC + c
    rs = s * RT
    _stage_slab(ei_hbm, 1, dst_v, wid)
    pltpu.sync_copy(ones_hbm, ones_v)
    pltpu.sync_copy(zrow_hbm, z_v)
    pltpu.sync_copy(z_v, deg_sh.at[pl.ds(rs, RT)])
    plsc.subcore_barrier()

    Q = 8
    for j in range(Q):
        pltpu.async_copy(ones_v, deg_sh.at[dst_v.at[j]], ssem, add=True)

    def body(ch, carry):
        pltpu.make_async_copy(ones_v, deg_sh.at[dst_v.at[ch]], ssem).wait()
        pltpu.async_copy(ones_v, deg_sh.at[dst_v.at[ch + Q]], ssem, add=True)
        return carry

    lax.fori_loop(0, C - Q, body, 0)
    for j in range(C - Q, C):
        pltpu.make_async_copy(ones_v, deg_sh.at[dst_v.at[j]], ssem).wait()
    plsc.subcore_barrier()
    pltpu.sync_copy(deg_sh.at[pl.ds(rs, RT)], out_hbm.at[c, pl.ds(rs, RT)])


@functools.partial(
    pl.kernel,
    out_type=jax.ShapeDtypeStruct((NC, NP, DG), bf16),
    mesh=_mesh,
    scratch_types=[
        pltpu.VMEM((C, K), jnp.int32),
        pltpu.VMEM((C, K), jnp.int32),
        pltpu.VMEM((K, DG), bf16),
        pltpu.VMEM((K, DG), bf16),
        pltpu.VMEM((K, DG), bf16),
        pltpu.VMEM((K, DG), bf16),
        pltpu.VMEM_SHARED((NP, DG), bf16),
        pltpu.SemaphoreType.DMA,
        pltpu.SemaphoreType.DMA,
        pltpu.SemaphoreType.DMA,
        pltpu.SemaphoreType.DMA,
        pltpu.SemaphoreType.DMA,
        pltpu.SemaphoreType.DMA,
        pltpu.SemaphoreType.DMA,
        pltpu.SemaphoreType.DMA,
    ],
    compiler_params=pltpu.CompilerParams(use_tc_tiling_on_sc=False),
)
def _scat_call(g_hbm, ei_hbm, zrows_hbm, out_hbm,
               src_v, dst_v, rows0, rows1, rows2, rows3, acc_sh,
               gs0, gs1, gs2, gs3, ss0, ss1, ss2, ss3):
    c = lax.axis_index("c")
    s = lax.axis_index("s")
    wid = s * NC + c
    rs = s * RT
    rows = (rows0, rows1, rows2, rows3)
    gsem = (gs0, gs1, gs2, gs3)
    ssem = (ss0, ss1, ss2, ss3)
    _stage_slab(ei_hbm, 0, src_v, wid)
    _stage_slab(ei_hbm, 1, dst_v, wid)

    @pl.when(c == 0)
    def _():
        pltpu.sync_copy(g_hbm.at[pl.ds(rs, RT)], acc_sh.at[pl.ds(rs, RT)])

    @pl.when(c != 0)
    def _():
        pltpu.sync_copy(zrows_hbm, acc_sh.at[pl.ds(rs, RT)])

    plsc.subcore_barrier()

    pltpu.async_copy(g_hbm.at[src_v.at[0]], rows0, gs0)
    pltpu.async_copy(g_hbm.at[src_v.at[1]], rows1, gs1)

    def body(i, carry):
        for b in range(4):
            ch = 4 * i + b
            bb = (b + 2) % 4

            @pl.when(ch + 2 < C)
            def _():
                @pl.when(ch >= 2)
                def _():
                    pltpu.make_async_copy(
                        rows[bb], acc_sh.at[dst_v.at[ch - 2]], ssem[bb]).wait()

                pltpu.async_copy(g_hbm.at[src_v.at[ch + 2]], rows[bb], gsem[bb])

            pltpu.make_async_copy(g_hbm.at[src_v.at[ch]], rows[b], gsem[b]).wait()
            pltpu.async_copy(rows[b], acc_sh.at[dst_v.at[ch]], ssem[b], add=True)

        return carry

    lax.fori_loop(0, C // 4, body, 0)
    for ch in range(C - 4, C):
        b = ch % 4
        pltpu.make_async_copy(rows[b], acc_sh.at[dst_v.at[ch]], ssem[b]).wait()
    plsc.subcore_barrier()
    pltpu.sync_copy(acc_sh.at[pl.ds(rs, RT)], out_hbm.at[c, pl.ds(rs, RT)])


def _tc1a_body(x_ref, W0_ref, b0_ref, h0_ref):
    h0_ref[:N] = jnp.dot(x_ref[:, DG:], W0_ref[...],
                         preferred_element_type=f32) + b0_ref[...]
    h0_ref[N:] = jnp.broadcast_to(b0_ref[...], (NP - N, DG))


def _tc1b_body(degT_ref, h0_ref, g0_ref, dinv_ref):
    deg = degT_ref[:, 0:1] + degT_ref[:, 1:2] + 1.0
    dinv = lax.rsqrt(deg)
    g0_ref[...] = (h0_ref[...] * dinv).astype(bf16)
    dinv_ref[...] = dinv


def _tc2_body(x_ref, acc_ref, dinv_ref, W1_ref, b1_ref, y0_ref, g1_ref):
    accs = acc_ref[0, :N].astype(f32) + acc_ref[1, :N].astype(f32)
    fm = jnp.maximum(accs * dinv_ref[:N], 0.0)
    y0 = x_ref[:, :DG] + fm
    y0_ref[...] = y0
    h1 = jnp.dot(y0, W1_ref[...], preferred_element_type=f32) + b1_ref[...]
    g1_ref[:N] = (h1 * dinv_ref[:N]).astype(bf16)
    g1_ref[N:] = (jnp.broadcast_to(b1_ref[...], (NP - N, DG))
                  * dinv_ref[N:]).astype(bf16)


def _tc3_body(x_ref, acc_ref, dinv_ref, y0_ref, out_ref):
    accs = acc_ref[0, :N].astype(f32) + acc_ref[1, :N].astype(f32)
    fm = jnp.maximum(accs * dinv_ref[:N], 0.0)
    out_ref[:, :DG] = y0_ref[...]
    out_ref[:, DG:] = x_ref[:, DG:] + fm


_tc1a = pl.pallas_call(
    _tc1a_body,
    out_shape=jax.ShapeDtypeStruct((NP, DG), f32),
)

_tc1b = pl.pallas_call(
    _tc1b_body,
    out_shape=[jax.ShapeDtypeStruct((NP, DG), bf16),
               jax.ShapeDtypeStruct((NP, 1), f32)],
)

_tc2 = pl.pallas_call(
    _tc2_body,
    out_shape=[jax.ShapeDtypeStruct((N, DG), f32),
               jax.ShapeDtypeStruct((NP, DG), bf16)],
)

_tc3 = pl.pallas_call(
    _tc3_body,
    out_shape=jax.ShapeDtypeStruct((N, D), f32),
)


def kernel(x, edge_index, W0, b0, W1, b1):
    x = x.astype(f32)
    ei3 = edge_index.astype(jnp.int32).reshape(2, TCH, K)

    ones_k = jnp.ones((K,), f32)
    zrow = jnp.zeros((RT,), f32)
    zrows = jnp.zeros((RT, DG), bf16)

    degp = _deg_call(ei3, ones_k, zrow)
    degT = degp.T

    h0 = _tc1a(x, W0, b0.reshape(1, DG))
    g0, dinv = _tc1b(degT, h0)
    acc0 = _scat_call(g0, ei3, zrows)
    y0, g1 = _tc2(x, acc0, dinv, W1, b1.reshape(1, DG))
    acc1 = _scat_call(g1, ei3, zrows)
    return _tc3(x, acc1, dinv, y0)


# speedup vs baseline: 53.8800x; 1.0083x over previous
import functools

import jax
import jax.numpy as jnp
from jax import lax
from jax.experimental import pallas as pl
from jax.experimental.pallas import tpu as pltpu
from jax.experimental.pallas import tpu_sc as plsc

N = 10000
D = 128
DG = 64
E = 320000

NC = 2
NS = 16
NT = NC * NS
NP = 10240
RT = NP // NS
K = 128
TCH = E // K
CB = TCH // NT
REM = TCH - CB * NT
C = CB + 2
PAD_ROWS = NP - N

f32 = jnp.float32
bf16 = jnp.bfloat16

_mesh = plsc.VectorSubcoreMesh(
    core_axis_name="c", subcore_axis_name="s", num_cores=NC, num_subcores=NS)


def _stage_slab(ei_hbm, which, slab_v, wid):
    base = wid * CB + jnp.minimum(wid, REM)
    pltpu.sync_copy(ei_hbm.at[which, pl.ds(base, CB)], slab_v.at[pl.ds(0, CB)])

    @pl.when(wid < REM)
    def _():
        pltpu.sync_copy(ei_hbm.at[which, pl.ds(base + CB, 1)],
                        slab_v.at[pl.ds(CB, 1)])

    def fill_row(r):
        for j in range(K // 16):
            vec = N + ((wid * K + r * 64 + j * 16
                        + lax.iota(jnp.int32, 16)) % PAD_ROWS)
            slab_v[r, pl.ds(j * 16, 16)] = vec

    @pl.when(wid >= REM)
    def _():
        fill_row(CB)

    fill_row(CB + 1)


@functools.partial(
    pl.kernel,
    out_type=jax.ShapeDtypeStruct((NC, NP), f32),
    mesh=_mesh,
    scratch_types=[
        pltpu.VMEM((C, K), jnp.int32),
        pltpu.VMEM((K,), f32),
        pltpu.VMEM((RT,), f32),
        pltpu.VMEM_SHARED((NP,), f32),
        pltpu.SemaphoreType.DMA,
    ],
    compiler_params=pltpu.CompilerParams(use_tc_tiling_on_sc=False),
)
def _deg_call(ei_hbm, ones_hbm, zrow_hbm, out_hbm, dst_v, ones_v, z_v,
              deg_sh, ssem):
    c = lax.axis_index("c")
    s = lax.axis_index("s")
    wid = s * NC + c
    rs = s * RT
    _stage_slab(ei_hbm, 1, dst_v, wid)
    pltpu.sync_copy(ones_hbm, ones_v)
    pltpu.sync_copy(zrow_hbm, z_v)
    pltpu.sync_copy(z_v, deg_sh.at[pl.ds(rs, RT)])
    plsc.subcore_barrier()

    Q = 8
    for j in range(Q):
        pltpu.async_copy(ones_v, deg_sh.at[dst_v.at[j]], ssem, add=True)

    def body(ch, carry):
        pltpu.make_async_copy(ones_v, deg_sh.at[dst_v.at[ch]], ssem).wait()
        pltpu.async_copy(ones_v, deg_sh.at[dst_v.at[ch + Q]], ssem, add=True)
        return carry

    lax.fori_loop(0, C - Q, body, 0)
    for j in range(C - Q, C):
        pltpu.make_async_copy(ones_v, deg_sh.at[dst_v.at[j]], ssem).wait()
    plsc.subcore_barrier()
    pltpu.sync_copy(deg_sh.at[pl.ds(rs, RT)], out_hbm.at[c, pl.ds(rs, RT)])


@functools.partial(
    pl.kernel,
    out_type=jax.ShapeDtypeStruct((NC, NP, DG), bf16),
    mesh=_mesh,
    scratch_types=[
        pltpu.VMEM((C, K), jnp.int32),
        pltpu.VMEM((C, K), jnp.int32),
        pltpu.VMEM((K, DG), bf16),
        pltpu.VMEM((K, DG), bf16),
        pltpu.VMEM((K, DG), bf16),
        pltpu.VMEM((K, DG), bf16),
        pltpu.VMEM((K, DG), bf16),
        pltpu.VMEM_SHARED((NP, DG), bf16),
        pltpu.SemaphoreType.DMA,
        pltpu.SemaphoreType.DMA,
        pltpu.SemaphoreType.DMA,
        pltpu.SemaphoreType.DMA,
        pltpu.SemaphoreType.DMA,
        pltpu.SemaphoreType.DMA,
        pltpu.SemaphoreType.DMA,
        pltpu.SemaphoreType.DMA,
        pltpu.SemaphoreType.DMA,
        pltpu.SemaphoreType.DMA,
    ],
    compiler_params=pltpu.CompilerParams(use_tc_tiling_on_sc=False),
)
def _scat_call(g_hbm, ei_hbm, zrows_hbm, out_hbm,
               src_v, dst_v, rows0, rows1, rows2, rows3, rows4, acc_sh,
               gs0, gs1, gs2, gs3, gs4, ss0, ss1, ss2, ss3, ss4):
    c = lax.axis_index("c")
    s = lax.axis_index("s")
    wid = s * NC + c
    rs = s * RT
    rows = (rows0, rows1, rows2, rows3, rows4)
    gsem = (gs0, gs1, gs2, gs3, gs4)
    ssem = (ss0, ss1, ss2, ss3, ss4)
    _stage_slab(ei_hbm, 0, src_v, wid)
    _stage_slab(ei_hbm, 1, dst_v, wid)

    @pl.when(c == 0)
    def _():
        pltpu.sync_copy(g_hbm.at[pl.ds(rs, RT)], acc_sh.at[pl.ds(rs, RT)])

    @pl.when(c != 0)
    def _():
        pltpu.sync_copy(zrows_hbm, acc_sh.at[pl.ds(rs, RT)])

    plsc.subcore_barrier()

    pltpu.async_copy(g_hbm.at[src_v.at[0]], rows0, gs0)
    pltpu.async_copy(g_hbm.at[src_v.at[1]], rows1, gs1)
    pltpu.async_copy(g_hbm.at[src_v.at[2]], rows2, gs2)

    def body(i, carry):
        for b in range(5):
            ch = 5 * i + b
            bb = (b + 3) % 5

            @pl.when(ch + 3 < C)
            def _():
                @pl.when(ch >= 2)
                def _():
                    pltpu.make_async_copy(
                        rows[bb], acc_sh.at[dst_v.at[ch - 2]], ssem[bb]).wait()

                pltpu.async_copy(g_hbm.at[src_v.at[ch + 3]], rows[bb], gsem[bb])

            pltpu.make_async_copy(g_hbm.at[src_v.at[ch]], rows[b], gsem[b]).wait()
            pltpu.async_copy(rows[b], acc_sh.at[dst_v.at[ch]], ssem[b], add=True)

        return carry

    lax.fori_loop(0, C // 5, body, 0)
    for ch in range(C - 5, C):
        b = ch % 5
        pltpu.make_async_copy(rows[b], acc_sh.at[dst_v.at[ch]], ssem[b]).wait()
    plsc.subcore_barrier()
    pltpu.sync_copy(acc_sh.at[pl.ds(rs, RT)], out_hbm.at[c, pl.ds(rs, RT)])


BG = 1280
GN = NP // BG

_b_rows64 = pl.BlockSpec((BG, DG), lambda i: (i, 0))
_b_x = pl.BlockSpec((BG, D), lambda i: (i, 0))
_b_acc = pl.BlockSpec((NC, BG, DG), lambda i: (0, i, 0))
_b_dinv = pl.BlockSpec((BG, 1), lambda i: (i, 0))
_b_w = pl.BlockSpec((DG, DG), lambda i: (0, 0))
_b_b = pl.BlockSpec((1, DG), lambda i: (0, 0))


def _tc1a_body(x_ref, W0_ref, b0_ref, h0_ref):
    h0_ref[...] = jnp.dot(x_ref[:, DG:], W0_ref[...],
                          preferred_element_type=f32) + b0_ref[...]


def _tc1b_body(degT_ref, h0_ref, g0_ref, dinv_ref):
    deg = degT_ref[:, 0:1] + degT_ref[:, 1:2] + 1.0
    dinv = lax.rsqrt(deg)
    g0_ref[...] = (h0_ref[...] * dinv).astype(bf16)
    dinv_ref[...] = dinv


def _tc2_body(x_ref, acc_ref, dinv_ref, W1_ref, b1_ref, y0_ref, g1_ref):
    accs = acc_ref[0].astype(f32) + acc_ref[1].astype(f32)
    fm = jnp.maximum(accs * dinv_ref[...], 0.0)
    y0 = x_ref[:, :DG] + fm
    y0_ref[...] = y0
    h1 = jnp.dot(y0, W1_ref[...], preferred_element_type=f32) + b1_ref[...]
    g1_ref[...] = (h1 * dinv_ref[...]).astype(bf16)


def _tc3_body(x_ref, acc_ref, dinv_ref, y0_ref, out_ref):
    accs = acc_ref[0].astype(f32) + acc_ref[1].astype(f32)
    fm = jnp.maximum(accs * dinv_ref[...], 0.0)
    out_ref[:, :DG] = y0_ref[...]
    out_ref[:, DG:] = x_ref[:, DG:] + fm


_tc1a = pl.pallas_call(
    _tc1a_body,
    grid=(GN,),
    in_specs=[_b_x, _b_w, _b_b],
    out_specs=_b_rows64,
    out_shape=jax.ShapeDtypeStruct((NP, DG), f32),
)

_tc1b = pl.pallas_call(
    _tc1b_body,
    grid=(GN,),
    in_specs=[pl.BlockSpec((BG, NC), lambda i: (i, 0)), _b_rows64],
    out_specs=[_b_rows64, _b_dinv],
    out_shape=[jax.ShapeDtypeStruct((NP, DG), bf16),
               jax.ShapeDtypeStruct((NP, 1), f32)],
)

_tc2 = pl.pallas_call(
    _tc2_body,
    grid=(GN,),
    in_specs=[_b_x, _b_acc, _b_dinv, _b_w, _b_b],
    out_specs=[_b_rows64, _b_rows64],
    out_shape=[jax.ShapeDtypeStruct((N, DG), f32),
               jax.ShapeDtypeStruct((NP, DG), bf16)],
)

_tc3 = pl.pallas_call(
    _tc3_body,
    grid=(GN,),
    in_specs=[_b_x, _b_acc, _b_dinv, _b_rows64],
    out_specs=pl.BlockSpec((BG, D), lambda i: (i, 0)),
    out_shape=jax.ShapeDtypeStruct((N, D), f32),
)


def kernel(x, edge_index, W0, b0, W1, b1):
    x = x.astype(f32)
    ei3 = edge_index.astype(jnp.int32).reshape(2, TCH, K)

    ones_k = jnp.ones((K,), f32)
    zrow = jnp.zeros((RT,), f32)
    zrows = jnp.zeros((RT, DG), bf16)

    degp = _deg_call(ei3, ones_k, zrow)
    degT = degp.T

    h0 = _tc1a(x, W0, b0.reshape(1, DG))
    g0, dinv = _tc1b(degT, h0)
    acc0 = _scat_call(g0, ei3, zrows)
    y0, g1 = _tc2(x, acc0, dinv, W1, b1.reshape(1, DG))
    acc1 = _scat_call(g1, ei3, zrows)
    return _tc3(x, acc1, dinv, y0)


# speedup vs baseline: 55.4401x; 1.0290x over previous
import functools

import jax
import jax.numpy as jnp
from jax import lax
from jax.experimental import pallas as pl
from jax.experimental.pallas import tpu as pltpu
from jax.experimental.pallas import tpu_sc as plsc

N = 10000
D = 128
DG = 64
E = 320000

NC = 2
NS = 16
NT = NC * NS
NP = 10240
RT = NP // NS
K = 128
TCH = E // K
CB = TCH // NT
REM = TCH - CB * NT
C = CB + 2
PAD_ROWS = NP - N
NBUF = 8
LEAD = 4

f32 = jnp.float32
bf16 = jnp.bfloat16

_mesh = plsc.VectorSubcoreMesh(
    core_axis_name="c", subcore_axis_name="s", num_cores=NC, num_subcores=NS)


def _stage_slab(ei_hbm, which, slab_v, wid):
    base = wid * CB + jnp.minimum(wid, REM)
    pltpu.sync_copy(ei_hbm.at[which, pl.ds(base, CB)], slab_v.at[pl.ds(0, CB)])

    @pl.when(wid < REM)
    def _():
        pltpu.sync_copy(ei_hbm.at[which, pl.ds(base + CB, 1)],
                        slab_v.at[pl.ds(CB, 1)])

    def fill_row(r):
        for j in range(K // 16):
            vec = N + ((wid * K + r * 64 + j * 16
                        + lax.iota(jnp.int32, 16)) % PAD_ROWS)
            slab_v[r, pl.ds(j * 16, 16)] = vec

    @pl.when(wid >= REM)
    def _():
        fill_row(CB)

    fill_row(CB + 1)


@functools.partial(
    pl.kernel,
    out_type=jax.ShapeDtypeStruct((NC, NP), f32),
    mesh=_mesh,
    scratch_types=[
        pltpu.VMEM((C, K), jnp.int32),
        pltpu.VMEM((K,), f32),
        pltpu.VMEM((RT,), f32),
        pltpu.VMEM_SHARED((NP,), f32),
        pltpu.SemaphoreType.DMA,
    ],
    compiler_params=pltpu.CompilerParams(use_tc_tiling_on_sc=False),
)
def _deg_call(ei_hbm, ones_hbm, zrow_hbm, out_hbm, dst_v, ones_v, z_v,
              deg_sh, ssem):
    c = lax.axis_index("c")
    s = lax.axis_index("s")
    wid = s * NC + c
    rs = s * RT
    _stage_slab(ei_hbm, 1, dst_v, wid)
    pltpu.sync_copy(ones_hbm, ones_v)
    pltpu.sync_copy(zrow_hbm, z_v)
    pltpu.sync_copy(z_v, deg_sh.at[pl.ds(rs, RT)])
    plsc.subcore_barrier()

    Q = 8
    for j in range(Q):
        pltpu.async_copy(ones_v, deg_sh.at[dst_v.at[j]], ssem, add=True)

    def body(ch, carry):
        pltpu.make_async_copy(ones_v, deg_sh.at[dst_v.at[ch]], ssem).wait()
        pltpu.async_copy(ones_v, deg_sh.at[dst_v.at[ch + Q]], ssem, add=True)
        return carry

    lax.fori_loop(0, C - Q, body, 0)
    for j in range(C - Q, C):
        pltpu.make_async_copy(ones_v, deg_sh.at[dst_v.at[j]], ssem).wait()
    plsc.subcore_barrier()
    pltpu.sync_copy(deg_sh.at[pl.ds(rs, RT)], out_hbm.at[c, pl.ds(rs, RT)])


@functools.partial(
    pl.kernel,
    out_type=jax.ShapeDtypeStruct((NC, NP, DG), bf16),
    mesh=_mesh,
    scratch_types=[
        pltpu.VMEM((C, K), jnp.int32),
        pltpu.VMEM((C, K), jnp.int32),
        *([pltpu.VMEM((K, DG), bf16)] * NBUF),
        pltpu.VMEM_SHARED((NP, DG), bf16),
        *([pltpu.SemaphoreType.DMA] * (2 * NBUF)),
    ],
    compiler_params=pltpu.CompilerParams(use_tc_tiling_on_sc=False),
)
def _scat_call(g_hbm, ei_hbm, zrows_hbm, out_hbm, *refs):
    src_v, dst_v = refs[0], refs[1]
    rows = refs[2:2 + NBUF]
    acc_sh = refs[2 + NBUF]
    gsem = refs[3 + NBUF:3 + 2 * NBUF]
    ssem = refs[3 + 2 * NBUF:3 + 3 * NBUF]
    c = lax.axis_index("c")
    s = lax.axis_index("s")
    wid = s * NC + c
    rs = s * RT
    _stage_slab(ei_hbm, 0, src_v, wid)
    _stage_slab(ei_hbm, 1, dst_v, wid)

    @pl.when(c == 0)
    def _():
        pltpu.sync_copy(g_hbm.at[pl.ds(rs, RT)], acc_sh.at[pl.ds(rs, RT)])

    @pl.when(c != 0)
    def _():
        pltpu.sync_copy(zrows_hbm, acc_sh.at[pl.ds(rs, RT)])

    plsc.subcore_barrier()

    for j in range(LEAD):
        pltpu.async_copy(g_hbm.at[src_v.at[j]], rows[j], gsem[j])

    SLACK = NBUF - LEAD

    def body(i, carry):
        for b in range(NBUF):
            ch = NBUF * i + b
            bb = (b + LEAD) % NBUF

            @pl.when(ch + LEAD < C)
            def _():
                @pl.when(ch >= SLACK)
                def _():
                    pltpu.make_async_copy(
                        rows[bb], acc_sh.at[dst_v.at[ch - SLACK]],
                        ssem[bb]).wait()

                pltpu.async_copy(g_hbm.at[src_v.at[ch + LEAD]],
                                 rows[bb], gsem[bb])

            pltpu.make_async_copy(g_hbm.at[src_v.at[ch]], rows[b], gsem[b]).wait()
            pltpu.async_copy(rows[b], acc_sh.at[dst_v.at[ch]], ssem[b], add=True)

        return carry

    lax.fori_loop(0, C // NBUF, body, 0)
    for ch in range(C - NBUF, C):
        b = ch % NBUF
        pltpu.make_async_copy(rows[b], acc_sh.at[dst_v.at[ch]], ssem[b]).wait()
    plsc.subcore_barrier()
    pltpu.sync_copy(acc_sh.at[pl.ds(rs, RT)], out_hbm.at[c, pl.ds(rs, RT)])


def _tc1a_body(x_ref, W0_ref, b0_ref, h0_ref):
    h0_ref[:N] = jnp.dot(x_ref[:, DG:], W0_ref[...],
                         preferred_element_type=f32) + b0_ref[...]
    h0_ref[N:] = jnp.broadcast_to(b0_ref[...], (NP - N, DG))


def _tc1b_body(degT_ref, h0_ref, g0_ref, dinv_ref):
    deg = degT_ref[:, 0:1] + degT_ref[:, 1:2] + 1.0
    dinv = lax.rsqrt(deg)
    g0_ref[...] = (h0_ref[...] * dinv).astype(bf16)
    dinv_ref[...] = dinv


def _tc2_body(x_ref, acc_ref, dinv_ref, W1_ref, b1_ref, y0_ref, g1_ref):
    accs = acc_ref[0, :N].astype(f32) + acc_ref[1, :N].astype(f32)
    fm = jnp.maximum(accs * dinv_ref[:N], 0.0)
    y0 = x_ref[:, :DG] + fm
    y0_ref[...] = y0
    h1 = jnp.dot(y0, W1_ref[...], preferred_element_type=f32) + b1_ref[...]
    g1_ref[:N] = (h1 * dinv_ref[:N]).astype(bf16)
    g1_ref[N:] = (jnp.broadcast_to(b1_ref[...], (NP - N, DG))
                  * dinv_ref[N:]).astype(bf16)


def _tc3_body(x_ref, acc_ref, dinv_ref, y0_ref, out_ref):
    accs = acc_ref[0, :N].astype(f32) + acc_ref[1, :N].astype(f32)
    fm = jnp.maximum(accs * dinv_ref[:N], 0.0)
    out_ref[:, :DG] = y0_ref[...]
    out_ref[:, DG:] = x_ref[:, DG:] + fm


_tc1a = pl.pallas_call(
    _tc1a_body,
    out_shape=jax.ShapeDtypeStruct((NP, DG), f32),
)

_tc1b = pl.pallas_call(
    _tc1b_body,
    out_shape=[jax.ShapeDtypeStruct((NP, DG), bf16),
               jax.ShapeDtypeStruct((NP, 1), f32)],
)

_tc2 = pl.pallas_call(
    _tc2_body,
    out_shape=[jax.ShapeDtypeStruct((N, DG), f32),
               jax.ShapeDtypeStruct((NP, DG), bf16)],
)

_tc3 = pl.pallas_call(
    _tc3_body,
    out_shape=jax.ShapeDtypeStruct((N, D), f32),
)


def kernel(x, edge_index, W0, b0, W1, b1):
    x = x.astype(f32)
    ei3 = edge_index.astype(jnp.int32).reshape(2, TCH, K)

    ones_k = jnp.ones((K,), f32)
    zrow = jnp.zeros((RT,), f32)
    zrows = jnp.zeros((RT, DG), bf16)

    degp = _deg_call(ei3, ones_k, zrow)
    degT = degp.T

    h0 = _tc1a(x, W0, b0.reshape(1, DG))
    g0, dinv = _tc1b(degT, h0)
    acc0 = _scat_call(g0, ei3, zrows)
    y0, g1 = _tc2(x, acc0, dinv, W1, b1.reshape(1, DG))
    acc1 = _scat_call(g1, ei3, zrows)
    return _tc3(x, acc1, dinv, y0)
